# baseline (speedup 1.0000x reference)
import functools

import jax
import jax.numpy as jnp
from jax import lax
from jax.experimental import pallas as pl
from jax.experimental.pallas import tpu as pltpu
from jax.experimental.pallas import tpu_sc as plsc

DICT_SIZE = 1000
_NC = 2
_NS = 16
_W = 32


def _sc_onehot_body(S, idx_hbm, out_hbm, idx_v, buf, sem):
    D = DICT_SIZE
    wid = lax.axis_index("s") * _NC + lax.axis_index("c")
    col0 = wid * _W
    blk0 = (wid // 4) * 128
    sub0 = (wid % 4) * _W

    pltpu.sync_copy(idx_hbm.at[:, pl.ds(blk0, 128)], idx_v)

    zeros16 = jnp.zeros((16,), jnp.float32)
    ones16 = jnp.ones((16,), jnp.float32)
    lane = lax.iota(jnp.int32, 16)
    nfull = D // 16
    tail = D - nfull * 16
    tail_mask = lane < tail

    def _zero_row(r, carry):
        for c in range(nfull):
            buf[r, pl.ds(c * 16, 16)] = zeros16
        plsc.store_scatter(
            buf, [jnp.full((16,), r, jnp.int32), nfull * 16 + lane],
            zeros16, mask=tail_mask,
        )
        return carry

    lax.fori_loop(0, _W, _zero_row, 0)

    def _scatter(s, val16):
        for ck in range(_W // 16):
            idxs = idx_v[s, pl.ds(sub0 + ck * 16, 16)]
            plsc.store_scatter(buf, [lane + ck * 16, idxs], val16)

    def _step(s, carry):
        _scatter(s, ones16)
        pltpu.sync_copy(buf, out_hbm.at[s, pl.ds(col0, _W), :])
        _scatter(s, zeros16)
        return carry

    lax.fori_loop(0, S, _step, 0)


def kernel(input, teacher_forcing):
    S, B = input.shape
    idx = input.astype(jnp.int32)

    sc_call = pl.kernel(
        functools.partial(_sc_onehot_body, S),
        out_type=jax.ShapeDtypeStruct((S, B, DICT_SIZE), jnp.float32),
        mesh=plsc.VectorSubcoreMesh(core_axis_name="c", subcore_axis_name="s"),
        scratch_types=[
            pltpu.VMEM((S, 128), jnp.int32),
            pltpu.VMEM((_W, DICT_SIZE), jnp.float32),
            pltpu.SemaphoreType.DMA,
        ],
        compiler_params=pltpu.CompilerParams(needs_layout_passes=False),
    )
    p2 = sc_call(idx)
    return (p2, input)


# speedup vs baseline: 1.0723x; 1.0723x over previous
import jax
import jax.numpy as jnp
from jax.experimental import pallas as pl
from jax.experimental.pallas import tpu as pltpu

DICT_SIZE = 1000
_NBUF = 4


def _onehot_ring_kernel(S, B, inp_ref, out_ref, scratch, sems):
    i = pl.program_id(0)
    slot = jax.lax.rem(i, _NBUF)

    @pl.when(i >= _NBUF)
    def _():
        pltpu.make_async_copy(
            scratch.at[pl.ds(slot, 1)],
            out_ref.at[pl.ds(i - _NBUF, 1)],
            sems.at[slot],
        ).wait()

    idx = inp_ref[0, 0, :]
    d = jax.lax.broadcasted_iota(jnp.int32, (B, DICT_SIZE), 1)
    scratch[pl.ds(slot, 1)] = (d == idx[:, None]).astype(jnp.float32)[None]
    pltpu.make_async_copy(
        scratch.at[pl.ds(slot, 1)],
        out_ref.at[pl.ds(i, 1)],
        sems.at[slot],
    ).start()

    @pl.when(i == S - 1)
    def _():
        for k in range(_NBUF):
            bi = S - _NBUF + k
            pltpu.make_async_copy(
                scratch.at[pl.ds(bi % _NBUF, 1)],
                out_ref.at[pl.ds(bi, 1)],
                sems.at[bi % _NBUF],
            ).wait()


def kernel(input, teacher_forcing):
    S, B = input.shape
    inp3 = input.reshape(S, 1, B)
    p2 = pl.pallas_call(
        lambda *refs: _onehot_ring_kernel(S, B, *refs),
        grid=(S,),
        in_specs=[pl.BlockSpec((1, 1, B), lambda i: (i, 0, 0))],
        out_specs=pl.BlockSpec(memory_space=pl.ANY),
        out_shape=jax.ShapeDtypeStruct((S, B, DICT_SIZE), jnp.float32),
        scratch_shapes=[
            pltpu.VMEM((_NBUF, B, DICT_SIZE), jnp.float32),
            pltpu.SemaphoreType.DMA((_NBUF,)),
        ],
        compiler_params=pltpu.CompilerParams(
            dimension_semantics=("arbitrary",),
        ),
    )(inp3)
    return (p2, input)
